# Initial kernel scaffold; baseline (speedup 1.0000x reference)
import jax
import jax.numpy as jnp
from jax.experimental import pallas as pl


def kernel(x, N):
    raise NotImplementedError("write your pallas kernel here")


# speedup vs baseline: 1.6326x; 1.6326x over previous
import functools

import jax
import jax.numpy as jnp
from jax import lax
from jax.experimental import pallas as pl
from jax.experimental.pallas import tpu as pltpu
from jax.experimental.pallas import tpu_sc as plsc


def _normalize_body(n_ref, p_ref):
    np_ = n_ref[...] + 1.0
    p_ref[...] = np_ / jnp.sum(np_, axis=1, keepdims=True)


def _make_gather(total, A, n_workers, chunk):
    per_w = total // n_workers
    n_ch = per_w // chunk
    mesh = plsc.VectorSubcoreMesh(core_axis_name="c", subcore_axis_name="s")

    @functools.partial(
        pl.kernel,
        mesh=mesh,
        compiler_params=pltpu.CompilerParams(use_tc_tiling_on_sc=False),
        out_type=jax.ShapeDtypeStruct((total, A), jnp.float32),
        scratch_types=[
            pltpu.VMEM((chunk,), jnp.int32),
            pltpu.VMEM((chunk, A), jnp.float32),
            pltpu.SemaphoreType.DMA,
        ],
    )
    def gather_k(x_hbm, p_hbm, out_hbm, idx_v, rows_v, sem):
        wid = lax.axis_index("s") * 2 + lax.axis_index("c")
        base = wid * per_w
        for ch in range(n_ch):
            off = base + ch * chunk
            pltpu.sync_copy(x_hbm.at[pl.ds(off, chunk)], idx_v)
            pltpu.async_copy(p_hbm.at[idx_v], rows_v, sem).wait()
            pltpu.sync_copy(rows_v, out_hbm.at[pl.ds(off, chunk)])

    return gather_k


def kernel(x, N):
    B, H = x.shape
    A = N.shape[0]
    total = B * H

    p = pl.pallas_call(
        _normalize_body,
        out_shape=jax.ShapeDtypeStruct((A, A), jnp.float32),
    )(N.astype(jnp.float32))

    x_flat = x.reshape(total).astype(jnp.int32)
    gather_k = _make_gather(total, A, n_workers=32, chunk=3200)
    out = gather_k(x_flat, p)
    return out.reshape(B, H, A)


# speedup vs baseline: 10.7676x; 6.5954x over previous
import functools

import jax
import jax.numpy as jnp
from jax import lax
from jax.experimental import pallas as pl
from jax.experimental.pallas import tpu as pltpu
from jax.experimental.pallas import tpu_sc as plsc

_LANES = 16


def _norm_t_body(nt_ref, pt_ref):
    npt = nt_ref[...] + 1.0
    s = jnp.sum(npt, axis=0, keepdims=True)
    pt_ref[:, : npt.shape[0]] = npt / s


def _make_sc_gather(H, HP, B, A, n_workers):
    ipw = B // n_workers
    ng = ipw // _LANES
    mesh = plsc.VectorSubcoreMesh(core_axis_name="c", subcore_axis_name="s")

    @functools.partial(
        pl.kernel,
        mesh=mesh,
        compiler_params=pltpu.CompilerParams(needs_layout_passes=False),
        out_type=jax.ShapeDtypeStruct((H, A, B), jnp.float32),
        scratch_types=[
            pltpu.VMEM((A * 128,), jnp.float32),
            pltpu.VMEM((2, ipw), jnp.int32),
            pltpu.VMEM((2, A, ipw), jnp.float32),
            pltpu.SemaphoreType.DMA,
            pltpu.SemaphoreType.DMA,
            pltpu.SemaphoreType.DMA,
            pltpu.SemaphoreType.DMA,
            pltpu.SemaphoreType.DMA,
        ],
    )
    def gather_k(xp_hbm, tab_hbm, out_hbm, tab_v, idx_v, out_v, sem_t,
                 sem_i0, sem_i1, sem_o0, sem_o1):
        wid = lax.axis_index("s") * 2 + lax.axis_index("c")
        i0 = wid * ipw
        sem_i = (sem_i0, sem_i1)
        sem_o = (sem_o0, sem_o1)

        pltpu.async_copy(tab_hbm, tab_v, sem_t).wait()
        for b in range(2):
            pltpu.async_copy(
                xp_hbm.at[b, pl.ds(i0, ipw)], idx_v.at[b], sem_i[b])

        def j_body(j0, carry):
            for b in range(2):
                j = j0 + b
                pltpu.make_async_copy(
                    xp_hbm.at[j, pl.ds(i0, ipw)], idx_v.at[b], sem_i[b]
                ).wait()

                @pl.when(j0 >= 2)
                def _wait_out():
                    pltpu.make_async_copy(
                        out_v.at[b],
                        out_hbm.at[j].at[:, pl.ds(i0, ipw)],
                        sem_o[b],
                    ).wait()

                for g in range(ng):
                    idx_g = idx_v[b, pl.ds(_LANES * g, _LANES)]
                    for c in range(A):
                        out_v[b, c, pl.ds(_LANES * g, _LANES)] = (
                            plsc.load_gather(
                                tab_v, [idx_g + jnp.int32(128 * c)]))

                pltpu.async_copy(
                    out_v.at[b],
                    out_hbm.at[j].at[:, pl.ds(i0, ipw)],
                    sem_o[b],
                )
                jn = jnp.minimum(j + 2, HP - 1)
                pltpu.async_copy(
                    xp_hbm.at[jn, pl.ds(i0, ipw)], idx_v.at[b], sem_i[b])
            return carry

        lax.fori_loop(0, H // 2, lambda t, c: j_body(2 * t, c), 0,
                      unroll=False)

        for b in range(2):
            pltpu.make_async_copy(
                xp_hbm.at[HP - 1, pl.ds(i0, ipw)], idx_v.at[b], sem_i[b]
            ).wait()
            pltpu.make_async_copy(
                out_v.at[b],
                out_hbm.at[H - 2 + b].at[:, pl.ds(i0, ipw)],
                sem_o[b],
            ).wait()

    return gather_k


def kernel(x, N):
    B, H = x.shape
    A = N.shape[0]
    HP = (H + 7) // 8 * 8

    pT = pl.pallas_call(
        _norm_t_body,
        out_shape=jax.ShapeDtypeStruct((A, 128), jnp.float32),
    )(N.astype(jnp.float32).T)

    xp = jnp.zeros((HP, B), jnp.int32).at[:H].set(x.T.astype(jnp.int32))

    gather_k = _make_sc_gather(H, HP, B, A, n_workers=32)
    out = gather_k(xp, pT.reshape(A * 128))
    return out.transpose(2, 0, 1)


# speedup vs baseline: 14.9411x; 1.3876x over previous
import functools

import jax
import jax.numpy as jnp
from jax import lax
from jax.experimental import pallas as pl
from jax.experimental.pallas import tpu as pltpu
from jax.experimental.pallas import tpu_sc as plsc

_LANES = 16


def _norm_t_body(nt_ref, pt_ref):
    npt = nt_ref[...] + 1.0
    s = jnp.sum(npt, axis=0, keepdims=True)
    pt_ref[:, : npt.shape[0]] = npt / s


def _make_sc_gather(H, HP, B, A, n_workers):
    ipw = B // n_workers
    ng = ipw // _LANES
    mesh = plsc.VectorSubcoreMesh(core_axis_name="c", subcore_axis_name="s")

    @functools.partial(
        pl.kernel,
        mesh=mesh,
        compiler_params=pltpu.CompilerParams(needs_layout_passes=False),
        out_type=jax.ShapeDtypeStruct((H, A, B), jnp.float32),
        scratch_types=[
            pltpu.VMEM((A * 128,), jnp.float32),
            pltpu.VMEM((2, ipw), jnp.int32),
            pltpu.VMEM((2, A, ipw), jnp.float32),
            pltpu.SemaphoreType.DMA,
            pltpu.SemaphoreType.DMA,
            pltpu.SemaphoreType.DMA,
            pltpu.SemaphoreType.DMA,
            pltpu.SemaphoreType.DMA,
        ],
    )
    def gather_k(xp_hbm, tab_hbm, out_hbm, tab_v, idx_v, out_v, sem_t,
                 sem_i0, sem_i1, sem_o0, sem_o1):
        wid = lax.axis_index("s") * 2 + lax.axis_index("c")
        i0 = wid * ipw
        sem_i = (sem_i0, sem_i1)
        sem_o = (sem_o0, sem_o1)

        pltpu.async_copy(tab_hbm, tab_v, sem_t).wait()
        for b in range(2):
            pltpu.async_copy(
                xp_hbm.at[b, pl.ds(i0, ipw)], idx_v.at[b], sem_i[b])

        def j_body(j0, carry):
            for b in range(2):
                j = j0 + b
                pltpu.make_async_copy(
                    xp_hbm.at[j, pl.ds(i0, ipw)], idx_v.at[b], sem_i[b]
                ).wait()

                @pl.when(j0 >= 2)
                def _wait_out():
                    pltpu.make_async_copy(
                        out_v.at[b],
                        out_hbm.at[j].at[:, pl.ds(i0, ipw)],
                        sem_o[b],
                    ).wait()

                for g in range(ng):
                    idx_g = idx_v[b, pl.ds(_LANES * g, _LANES)]
                    for c in range(A):
                        out_v[b, c, pl.ds(_LANES * g, _LANES)] = (
                            plsc.load_gather(
                                tab_v.at[pl.ds(128 * c, 128)], [idx_g]))

                pltpu.async_copy(
                    out_v.at[b],
                    out_hbm.at[j].at[:, pl.ds(i0, ipw)],
                    sem_o[b],
                )
                jn = jnp.minimum(j + 2, HP - 1)
                pltpu.async_copy(
                    xp_hbm.at[jn, pl.ds(i0, ipw)], idx_v.at[b], sem_i[b])
            return carry

        lax.fori_loop(0, H // 2, lambda t, c: j_body(2 * t, c), 0,
                      unroll=False)

        for b in range(2):
            pltpu.make_async_copy(
                xp_hbm.at[HP - 1, pl.ds(i0, ipw)], idx_v.at[b], sem_i[b]
            ).wait()
            pltpu.make_async_copy(
                out_v.at[b],
                out_hbm.at[H - 2 + b].at[:, pl.ds(i0, ipw)],
                sem_o[b],
            ).wait()

    return gather_k


def kernel(x, N):
    B, H = x.shape
    A = N.shape[0]
    HP = (H + 7) // 8 * 8

    pT = pl.pallas_call(
        _norm_t_body,
        out_shape=jax.ShapeDtypeStruct((A, 128), jnp.float32),
    )(N.astype(jnp.float32).T)

    xp = jnp.zeros((HP, B), jnp.int32).at[:H].set(x.T.astype(jnp.int32))

    gather_k = _make_sc_gather(H, HP, B, A, n_workers=32)
    out = gather_k(xp, pT.reshape(A * 128))
    return out.transpose(2, 0, 1)
